# baseline (speedup 1.0000x reference)
import functools
import jax
import jax.numpy as jnp
from jax import lax
from jax.experimental import pallas as pl
from jax.experimental.pallas import tpu as pltpu
from jax.experimental.pallas import tpu_sc as plsc

N = 10000
D = 128
R = 4
B = 2
E = 80000

NSC = 2
NTILE = 16
RELS_PER_SC = R // NSC
EP_TILE = E // NTILE
CHUNK = 72
NCHUNK = 70
EP_PAD = NCHUNK * CHUNK
NPAD = 10240
STRIPE = NPAD // NTILE
XSTRIPE = N // NTILE
H = D // 2
NBUF = 7
NGROUP = NCHUNK // NBUF


def _sc_body(p, *refs):
  if p == 0:
    (xh_hbm, src_hbm, dst_hbm, zrow_hbm, zcnt_hbm, ones_hbm,
     sums_hbm, cnt_hbm,
     src_v, dst_v, bufs, ones_v,
     xs_sh, accum_sh, cnt_sh, gsem, ssem, csem) = refs
  else:
    (xh_hbm, src_hbm, dst_hbm, zrow_hbm,
     sums_hbm,
     src_v, dst_v, bufs,
     xs_sh, accum_sh, gsem, ssem) = refs
  c = lax.axis_index("c")
  s = lax.axis_index("s")
  if p == 0:
    pltpu.sync_copy(ones_hbm, ones_v)
  pltpu.sync_copy(xh_hbm.at[pl.ds(s * XSTRIPE, XSTRIPE), pl.ds(p * H, H)],
                  xs_sh.at[pl.ds(s * XSTRIPE, XSTRIPE)])
  for k in range(RELS_PER_SC):
    rel = c * RELS_PER_SC + k
    pltpu.sync_copy(zrow_hbm, accum_sh.at[pl.ds(s * STRIPE, STRIPE)])
    if p == 0:
      pltpu.sync_copy(zcnt_hbm, cnt_sh.at[pl.ds(s * STRIPE, STRIPE)])
    pltpu.sync_copy(src_hbm.at[rel, s], src_v)
    pltpu.sync_copy(dst_hbm.at[rel, s], dst_v)
    plsc.subcore_barrier()

    for b in range(NBUF):
      pltpu.async_copy(xs_sh.at[src_v.at[b]], bufs.at[b], gsem.at[b])

    def body(g, carry):
      base = g * NBUF
      for b in range(NBUF):
        ch = base + b
        pltpu.make_async_copy(xs_sh.at[src_v.at[ch]], bufs.at[b],
                              gsem.at[b]).wait()
        pltpu.async_copy(bufs.at[b], accum_sh.at[dst_v.at[ch]],
                         ssem.at[b], add=True)
        if p == 0:
          pltpu.async_copy(ones_v, cnt_sh.at[dst_v.at[ch]],
                           csem.at[b], add=True)
      for b in range(NBUF):
        ch = base + b

        @pl.when(g < NGROUP - 1)
        def _():
          pltpu.make_async_copy(bufs.at[b], accum_sh.at[dst_v.at[ch]],
                                ssem.at[b]).wait()
          if p == 0:
            pltpu.make_async_copy(ones_v, cnt_sh.at[dst_v.at[ch]],
                                  csem.at[b]).wait()
          pltpu.async_copy(xs_sh.at[src_v.at[ch + NBUF]], bufs.at[b],
                           gsem.at[b])
      return carry

    lax.fori_loop(0, NGROUP, body, 0)
    last = (NGROUP - 1) * NBUF
    for b in range(NBUF):
      pltpu.make_async_copy(bufs.at[b], accum_sh.at[dst_v.at[last + b]],
                            ssem.at[b]).wait()
      if p == 0:
        pltpu.make_async_copy(ones_v, cnt_sh.at[dst_v.at[last + b]],
                              csem.at[b]).wait()
    plsc.subcore_barrier()
    pltpu.sync_copy(accum_sh.at[pl.ds(s * STRIPE, STRIPE)],
                    sums_hbm.at[rel, pl.ds(s * STRIPE, STRIPE)])
    if p == 0:
      pltpu.sync_copy(cnt_sh.at[pl.ds(s * STRIPE, STRIPE)],
                      cnt_hbm.at[rel, pl.ds(s * STRIPE, STRIPE)])


@functools.cache
def _sc_aggregate_fn(p):
  sums_t = jax.ShapeDtypeStruct((R, NPAD, H), jnp.float32)
  cnt_t = jax.ShapeDtypeStruct((R, NPAD), jnp.float32)
  scratch = [
      pltpu.VMEM((NCHUNK, CHUNK), jnp.int32),
      pltpu.VMEM((NCHUNK, CHUNK), jnp.int32),
      pltpu.VMEM((NBUF, CHUNK, H), jnp.float32),
  ]
  if p == 0:
    scratch.append(pltpu.VMEM((CHUNK,), jnp.float32))
  scratch += [
      pltpu.VMEM_SHARED((N, H), jnp.float32),
      pltpu.VMEM_SHARED((NPAD, H), jnp.float32),
  ]
  if p == 0:
    scratch.append(pltpu.VMEM_SHARED((NPAD,), jnp.float32))
  scratch += [
      pltpu.SemaphoreType.DMA((NBUF,)),
      pltpu.SemaphoreType.DMA((NBUF,)),
  ]
  if p == 0:
    scratch.append(pltpu.SemaphoreType.DMA((NBUF,)))
  return pl.kernel(
      functools.partial(_sc_body, p),
      out_type=(sums_t, cnt_t) if p == 0 else sums_t,
      mesh=plsc.VectorSubcoreMesh(core_axis_name="c", subcore_axis_name="s",
                                  num_cores=NSC, num_subcores=NTILE),
      compiler_params=pltpu.CompilerParams(use_tc_tiling_on_sc=False),
      scratch_types=scratch,
  )


BLK = 1280


def _tc_body(p, sums_ref, cnt_ref, w_ref, wcomp_ref, extra_ref, out_ref):
  acc = jnp.zeros((BLK, D), jnp.float32)
  w0 = w_ref[0][p * H:(p + 1) * H]
  w1 = w_ref[1][p * H:(p + 1) * H]
  for r in range(R):
    wr = wcomp_ref[r, 0] * w0 + wcomp_ref[r, 1] * w1
    cnt = jnp.maximum(cnt_ref[r], 1.0)[:, None]
    mean = sums_ref[r] / cnt
    acc = acc + jnp.dot(mean, wr, preferred_element_type=jnp.float32)
  out_ref[...] = acc + extra_ref[0] if p == 0 else acc + extra_ref[...]


def _tc_combine(p, sums, cnt, weight, w_comp, extra):
  extra_spec = (pl.BlockSpec((1, D), lambda i: (0, 0)) if p == 0
                else pl.BlockSpec((BLK, D), lambda i: (i, 0)))
  return pl.pallas_call(
      functools.partial(_tc_body, p),
      grid=(pl.cdiv(N, BLK),),
      in_specs=[
          pl.BlockSpec((R, BLK, H), lambda i: (0, i, 0)),
          pl.BlockSpec((R, BLK), lambda i: (0, i)),
          pl.BlockSpec((B, D, D), lambda i: (0, 0, 0)),
          pl.BlockSpec(memory_space=pltpu.SMEM),
          extra_spec,
      ],
      out_specs=pl.BlockSpec((BLK, D), lambda i: (i, 0)),
      out_shape=jax.ShapeDtypeStruct((N, D), jnp.float32),
  )(sums, cnt, weight, w_comp, extra)


@jax.jit
def kernel(x, edge_index, weight, w_comp, h_bias):
  src = edge_index[:, 0, :].reshape(R, NTILE, EP_TILE)
  dst = edge_index[:, 1, :].reshape(R, NTILE, EP_TILE)
  pad = EP_PAD - EP_TILE
  src = jnp.pad(src, ((0, 0), (0, 0), (0, pad))).reshape(R, NTILE, NCHUNK, CHUNK)
  dst = jnp.pad(dst, ((0, 0), (0, 0), (0, pad)), constant_values=N)
  dst = dst.reshape(R, NTILE, NCHUNK, CHUNK)
  zrow = jnp.zeros((STRIPE, H), jnp.float32)
  zcnt = jnp.zeros((STRIPE,), jnp.float32)
  ones = jnp.ones((CHUNK,), jnp.float32)

  sums0, cnt = _sc_aggregate_fn(0)(x, src, dst, zrow, zcnt, ones)
  sums1 = _sc_aggregate_fn(1)(x, src, dst, zrow)
  h0 = _tc_combine(0, sums0, cnt, weight, w_comp, h_bias.reshape(1, D))
  return _tc_combine(1, sums1, cnt, weight, w_comp, h0)


# speedup vs baseline: 1.0151x; 1.0151x over previous
import functools
import jax
import jax.numpy as jnp
from jax import lax
from jax.experimental import pallas as pl
from jax.experimental.pallas import tpu as pltpu
from jax.experimental.pallas import tpu_sc as plsc

N = 10000
D = 128
R = 4
B = 2
E = 80000

NSC = 2
NTILE = 16
RELS_PER_SC = R // NSC
EP_TILE = E // NTILE
CHUNK = 72
NCHUNK = 70
EP_PAD = NCHUNK * CHUNK
NPAD = 10240
STRIPE = NPAD // NTILE
XSTRIPE = N // NTILE
H = D // 2
NBUF = 7
NGROUP = NCHUNK // NBUF


def _sc_body(xh_hbm, src_hbm, dst_hbm, zrow_hbm, zcnt_hbm, ones_hbm,
             sums_hbm, cnt_hbm,
             src_v, dst_v, bufs, ones_v,
             xs_sh, accum_sh, cnt_sh, gsem, ssem, csem):
  c = lax.axis_index("c")
  s = lax.axis_index("s")
  pltpu.sync_copy(ones_hbm, ones_v)
  for p in range(2):
    pltpu.sync_copy(xh_hbm.at[pl.ds(s * XSTRIPE, XSTRIPE), pl.ds(p * H, H)],
                    xs_sh.at[pl.ds(s * XSTRIPE, XSTRIPE)])
    for k in range(RELS_PER_SC):
      rel = c * RELS_PER_SC + k
      pltpu.sync_copy(zrow_hbm, accum_sh.at[pl.ds(s * STRIPE, STRIPE)])
      if p == 0:
        pltpu.sync_copy(zcnt_hbm, cnt_sh.at[pl.ds(s * STRIPE, STRIPE)])
      pltpu.sync_copy(src_hbm.at[rel, s], src_v)
      pltpu.sync_copy(dst_hbm.at[rel, s], dst_v)
      plsc.subcore_barrier()

      for b in range(NBUF):
        pltpu.async_copy(xs_sh.at[src_v.at[b]], bufs.at[b], gsem.at[b])

      def body(g, carry):
        base = g * NBUF
        for b in range(NBUF):
          ch = base + b
          pltpu.make_async_copy(xs_sh.at[src_v.at[ch]], bufs.at[b],
                                gsem.at[b]).wait()
          pltpu.async_copy(bufs.at[b], accum_sh.at[dst_v.at[ch]],
                           ssem.at[b], add=True)
          if p == 0:
            pltpu.async_copy(ones_v, cnt_sh.at[dst_v.at[ch]],
                             csem.at[b], add=True)
        for b in range(NBUF):
          ch = base + b

          @pl.when(g < NGROUP - 1)
          def _():
            pltpu.make_async_copy(bufs.at[b], accum_sh.at[dst_v.at[ch]],
                                  ssem.at[b]).wait()
            if p == 0:
              pltpu.make_async_copy(ones_v, cnt_sh.at[dst_v.at[ch]],
                                    csem.at[b]).wait()
            pltpu.async_copy(xs_sh.at[src_v.at[ch + NBUF]], bufs.at[b],
                             gsem.at[b])
        return carry

      lax.fori_loop(0, NGROUP, body, 0)
      last = (NGROUP - 1) * NBUF
      for b in range(NBUF):
        pltpu.make_async_copy(bufs.at[b], accum_sh.at[dst_v.at[last + b]],
                              ssem.at[b]).wait()
        if p == 0:
          pltpu.make_async_copy(ones_v, cnt_sh.at[dst_v.at[last + b]],
                                csem.at[b]).wait()
      plsc.subcore_barrier()
      pltpu.sync_copy(accum_sh.at[pl.ds(s * STRIPE, STRIPE)],
                      sums_hbm.at[p, rel, pl.ds(s * STRIPE, STRIPE)])
      if p == 0:
        pltpu.sync_copy(cnt_sh.at[pl.ds(s * STRIPE, STRIPE)],
                        cnt_hbm.at[rel, pl.ds(s * STRIPE, STRIPE)])


@functools.cache
def _sc_aggregate_fn():
  return pl.kernel(
      _sc_body,
      out_type=(
          jax.ShapeDtypeStruct((2, R, NPAD, H), jnp.float32),
          jax.ShapeDtypeStruct((R, NPAD), jnp.float32),
      ),
      mesh=plsc.VectorSubcoreMesh(core_axis_name="c", subcore_axis_name="s",
                                  num_cores=NSC, num_subcores=NTILE),
      compiler_params=pltpu.CompilerParams(use_tc_tiling_on_sc=False),
      scratch_types=[
          pltpu.VMEM((NCHUNK, CHUNK), jnp.int32),
          pltpu.VMEM((NCHUNK, CHUNK), jnp.int32),
          pltpu.VMEM((NBUF, CHUNK, H), jnp.float32),
          pltpu.VMEM((CHUNK,), jnp.float32),
          pltpu.VMEM_SHARED((N, H), jnp.float32),
          pltpu.VMEM_SHARED((NPAD, H), jnp.float32),
          pltpu.VMEM_SHARED((NPAD,), jnp.float32),
          pltpu.SemaphoreType.DMA((NBUF,)),
          pltpu.SemaphoreType.DMA((NBUF,)),
          pltpu.SemaphoreType.DMA((NBUF,)),
      ],
  )


BLK = 1280


def _tc_body(sums_ref, cnt_ref, w_ref, wcomp_ref, bias_ref, out_ref):
  acc = jnp.zeros((BLK, D), jnp.float32)
  w0 = w_ref[0]
  w1 = w_ref[1]
  for r in range(R):
    wr = wcomp_ref[r, 0] * w0 + wcomp_ref[r, 1] * w1
    cnt = jnp.maximum(cnt_ref[r], 1.0)[:, None]
    mean = jnp.concatenate([sums_ref[0, r], sums_ref[1, r]], axis=1) / cnt
    acc = acc + jnp.dot(mean, wr, preferred_element_type=jnp.float32)
  out_ref[...] = acc + bias_ref[0]


def _tc_combine(sums, cnt, weight, w_comp, h_bias):
  return pl.pallas_call(
      _tc_body,
      grid=(pl.cdiv(N, BLK),),
      in_specs=[
          pl.BlockSpec((2, R, BLK, H), lambda i: (0, 0, i, 0)),
          pl.BlockSpec((R, BLK), lambda i: (0, i)),
          pl.BlockSpec((B, D, D), lambda i: (0, 0, 0)),
          pl.BlockSpec(memory_space=pltpu.SMEM),
          pl.BlockSpec((1, D), lambda i: (0, 0)),
      ],
      out_specs=pl.BlockSpec((BLK, D), lambda i: (i, 0)),
      out_shape=jax.ShapeDtypeStruct((N, D), jnp.float32),
  )(sums, cnt, weight, w_comp, h_bias.reshape(1, D))


@jax.jit
def kernel(x, edge_index, weight, w_comp, h_bias):
  src = edge_index[:, 0, :].reshape(R, NTILE, EP_TILE)
  dst = edge_index[:, 1, :].reshape(R, NTILE, EP_TILE)
  pad = EP_PAD - EP_TILE
  src = jnp.pad(src, ((0, 0), (0, 0), (0, pad))).reshape(R, NTILE, NCHUNK, CHUNK)
  dst = jnp.pad(dst, ((0, 0), (0, 0), (0, pad)), constant_values=N)
  dst = dst.reshape(R, NTILE, NCHUNK, CHUNK)
  zrow = jnp.zeros((STRIPE, H), jnp.float32)
  zcnt = jnp.zeros((STRIPE,), jnp.float32)
  ones = jnp.ones((CHUNK,), jnp.float32)

  sums, cnt = _sc_aggregate_fn()(x, src, dst, zrow, zcnt, ones)
  return _tc_combine(sums, cnt, weight, w_comp, h_bias)
